# baseline (speedup 1.0000x reference)
import functools

import jax
import jax.numpy as jnp
from jax import lax
from jax.experimental import pallas as pl
from jax.experimental.pallas import tpu as pltpu
from jax.experimental.pallas import tpu_sc as plsc


def _build_pack(V, D, NC, NS):
    NW = NC * NS
    n_full = V // 128
    tail = V % 128
    per_w = n_full // NW
    extra = n_full % NW
    SA = 3

    mesh = plsc.VectorSubcoreMesh(core_axis_name="c", subcore_axis_name="s")

    @functools.partial(
        pl.kernel,
        mesh=mesh,
        out_type=jax.ShapeDtypeStruct(((V + 127) // 2 // 64 * 64, 128), jnp.float32),
        scratch_types=[
            pltpu.VMEM((SA, D, 133), jnp.float32),
            pltpu.VMEM((SA, D, 128), jnp.float32),
        ]
        + [pltpu.SemaphoreType.DMA] * (2 * SA),
        compiler_params=pltpu.CompilerParams(
            use_tc_tiling_on_sc=False, needs_layout_passes=False
        ),
    )
    def pack(t5_hbm, s2_hbm, abuf, bbuf, *sems):
        gsem = sems[:SA]
        wsem = sems[SA:]
        wid = lax.axis_index("s") * NC + lax.axis_index("c")
        j0 = wid * per_w + jnp.minimum(wid, extra)
        nj = per_w + jnp.where(wid < extra, 1, 0)

        lane = jax.lax.broadcasted_iota(jnp.int32, (16,), 0)
        dvs = [lane + 16 * k for k in range(D // 16)]

        def gdesc(j, s):
            return pltpu.make_async_copy(
                t5_hbm.at[:, pl.ds(j * 128, 128)],
                abuf.at[s, :, pl.ds(0, 128)],
                gsem[s],
            )

        def wdesc(j, s):
            return pltpu.make_async_copy(
                bbuf.at[s], s2_hbm.at[pl.ds(j * 64, 64)], wsem[s]
            )

        def transpose_block(s, nl):
            @plsc.parallel_loop(0, nl, step=1, unroll=4)
            def _(l):
                lfull = lane * 0 + l
                row = l // 2
                cb = (l % 2) * 64
                for k in range(D // 16):
                    vals = plsc.load_gather(abuf.at[s], [dvs[k], lfull])
                    bbuf[s, row, pl.ds(cb + k * 16, 16)] = vals

        for s in range(SA):
            @pl.when(s < nj)
            def _():
                gdesc(j0 + s, s).start()

        def body(t, carry):
            for s in range(SA):
                i = t * SA + s

                @pl.when(i < nj)
                def _():
                    gdesc(j0 + i, s).wait()

                    @pl.when(t > 0)
                    def _():
                        wdesc(0, s).wait()

                    transpose_block(s, 128)
                    wdesc(j0 + i, s).start()

                    @pl.when(i + SA < nj)
                    def _():
                        gdesc(j0 + i + SA, s).start()

            return carry

        n_rounds = (per_w + 1 + SA - 1) // SA
        lax.fori_loop(0, n_rounds, body, 0)
        for s in range(SA):
            @pl.when((nj >= SA) | (s < nj))
            def _():
                wdesc(0, s).wait()

        if tail:
            @pl.when(wid == NW - 1)
            def _():
                pltpu.sync_copy(
                    t5_hbm.at[:, pl.ds(n_full * 128, tail)],
                    abuf.at[0, :, pl.ds(0, tail)],
                )
                transpose_block(0, tail)
                pltpu.sync_copy(
                    bbuf.at[0, pl.ds(0, tail // 2)],
                    s2_hbm.at[pl.ds(n_full * 64, tail // 2)],
                )

    return pack


def _build_emb(H, B, V, D, NC, NS):
    NW = NC * NS
    BLK = B // NW
    CHUNK = BLK
    SLOTS = 4
    n_rounds = H // SLOTS
    assert H % SLOTS == 0 and BLK == 128

    mesh = plsc.VectorSubcoreMesh(core_axis_name="c", subcore_axis_name="s")

    @functools.partial(
        pl.kernel,
        mesh=mesh,
        out_type=jax.ShapeDtypeStruct((H, D // 8, NW, 8, BLK), jnp.float32),
        scratch_types=[
            pltpu.VMEM((H, BLK), jnp.int32),
            pltpu.VMEM((SLOTS, 2, BLK), jnp.int32),
            pltpu.VMEM((SLOTS, CHUNK, D // 2), jnp.float32),
            pltpu.VMEM((SLOTS, CHUNK, D // 2), jnp.float32),
            pltpu.VMEM((SLOTS, D // 8, 1, 8, BLK + 5), jnp.float32),
        ]
        + [pltpu.SemaphoreType.DMA] * (2 * SLOTS),
        compiler_params=pltpu.CompilerParams(
            use_tc_tiling_on_sc=False, needs_layout_passes=False
        ),
    )
    def emb(xt_hbm, s2q_hbm, out_hbm, idx_v, idq_v, rows_e, rows_o, tbuf_v, *sems):
        gsem = sems[:SLOTS]
        wsem = sems[SLOTS:]
        wid = lax.axis_index("s") * NC + lax.axis_index("c")
        base = wid * BLK
        pltpu.sync_copy(xt_hbm.at[:, pl.ds(base, BLK)], idx_v)

        lane = jax.lax.broadcasted_iota(jnp.int32, (16,), 0)
        lane_hi = lane // 8
        lane_lo = lane % 8
        zeros16 = lane * 0
        dhis = [lane_hi + 2 * k for k in range(D // 16)]

        def gdescs(h, j):
            return (
                pltpu.make_async_copy(
                    s2q_hbm.at[idq_v.at[j, 0]], rows_e.at[j], gsem[j]
                ),
                pltpu.make_async_copy(
                    s2q_hbm.at[idq_v.at[j, 1]], rows_o.at[j], gsem[j]
                ),
            )

        def wdesc(h, j):
            return pltpu.make_async_copy(
                tbuf_v.at[j, :, :, :, pl.ds(0, BLK)],
                out_hbm.at[h, pl.ds(0, D // 8), pl.ds(wid, 1)],
                wsem[j],
            )

        def stage_idx(h, j):
            @plsc.parallel_loop(0, BLK // 16, step=1, unroll=4)
            def _(g):
                v = idx_v[h, pl.ds(g * 16, 16)]
                v2 = v + v
                idq_v[j, 0, pl.ds(g * 16, 16)] = v2
                idq_v[j, 1, pl.ds(g * 16, 16)] = v2 + 1

        def fire(h, j):
            stage_idx(h, j)
            de, do = gdescs(h, j)
            de.start()
            do.start()

        for j in range(SLOTS):
            fire(j, j)

        def body(t, carry):
            for j in range(SLOTS):
                h = t * SLOTS + j
                de, do = gdescs(h, j)
                de.wait()
                do.wait()

                @pl.when(t > 0)
                def _():
                    wdesc(h - SLOTS, j).wait()

                @plsc.parallel_loop(0, CHUNK, step=1, unroll=8)
                def trans(b):
                    bfull = zeros16 + b
                    for k in range(D // 16):
                        src = rows_e if k < (D // 32) else rows_o
                        vals = src[j, b, pl.ds((k % (D // 32)) * 16, 16)]
                        plsc.store_scatter(
                            tbuf_v.at[j],
                            [dhis[k], zeros16, lane_lo, bfull],
                            vals,
                        )

                wdesc(h, j).start()

                @pl.when(h + SLOTS < H)
                def _():
                    fire(h + SLOTS, j)

            return carry

        lax.fori_loop(0, n_rounds, body, 0)
        for j in range(SLOTS):
            wdesc(H - SLOTS + j, j).wait()

    return emb


def kernel(x, table):
    Bt, H = x.shape
    V, D = table.shape
    info = plsc.get_sparse_core_info()
    pack = _build_pack(V, D, info.num_cores, info.num_subcores)
    emb = _build_emb(H, Bt, V, D, info.num_cores, info.num_subcores)
    s2 = pack(table.T)
    s2q = s2.reshape(s2.shape[0] * 4, D // 2)
    out5 = emb(x.T, s2q)
    return out5.transpose(2, 4, 0, 1, 3).reshape(Bt, H, D)


# speedup vs baseline: 5.6190x; 5.6190x over previous
import functools

import jax
import jax.numpy as jnp
from jax import lax
from jax.experimental import pallas as pl
from jax.experimental.pallas import tpu as pltpu
from jax.experimental.pallas import tpu_sc as plsc


def _build_pack(V, D, NC, NS):
    NW = NC * NS
    n_full = V // 128
    tail = V % 128
    per_w = n_full // NW
    extra = n_full % NW
    SA = 3

    mesh = plsc.VectorSubcoreMesh(core_axis_name="c", subcore_axis_name="s")

    @functools.partial(
        pl.kernel,
        mesh=mesh,
        out_type=jax.ShapeDtypeStruct(((V + 127) // 2 // 64 * 64, 128), jnp.float32),
        scratch_types=[
            pltpu.VMEM((SA, D, 133), jnp.float32),
            pltpu.VMEM((SA, D, 128), jnp.float32),
            pltpu.VMEM((D, tail if tail else 128), jnp.float32),
        ]
        + [pltpu.SemaphoreType.DMA] * (2 * SA),
        compiler_params=pltpu.CompilerParams(
            use_tc_tiling_on_sc=True, needs_layout_passes=False
        ),
    )
    def pack(t5_hbm, s2_hbm, abuf, bbuf, tbuf, *sems):
        gsem = sems[:SA]
        wsem = sems[SA:]
        wid = lax.axis_index("s") * NC + lax.axis_index("c")
        j0 = wid * per_w + jnp.minimum(wid, extra)
        nj = per_w + jnp.where(wid < extra, 1, 0)

        lane = jax.lax.broadcasted_iota(jnp.int32, (16,), 0)
        dvs = [lane + 16 * k for k in range(D // 16)]

        def gdesc(j, s):
            return pltpu.make_async_copy(
                t5_hbm.at[:, pl.ds(j * 128, 128)],
                abuf.at[s, :, pl.ds(0, 128)],
                gsem[s],
            )

        def wdesc(j, s):
            return pltpu.make_async_copy(
                bbuf.at[s], s2_hbm.at[pl.ds(j * 64, 64)], wsem[s]
            )

        def transpose_block(s, nl):
            @plsc.parallel_loop(0, nl, step=1, unroll=4)
            def _(l):
                lfull = lane * 0 + l
                row = l // 2
                cb = (l % 2) * 64
                for k in range(D // 16):
                    vals = plsc.load_gather(abuf.at[s], [dvs[k], lfull])
                    bbuf[s, row, pl.ds(cb + k * 16, 16)] = vals

        for s in range(SA):
            @pl.when(s < nj)
            def _():
                gdesc(j0 + s, s).start()

        def body(t, carry):
            for s in range(SA):
                i = t * SA + s

                @pl.when(i < nj)
                def _():
                    gdesc(j0 + i, s).wait()

                    @pl.when(t > 0)
                    def _():
                        wdesc(0, s).wait()

                    transpose_block(s, 128)
                    wdesc(j0 + i, s).start()

                    @pl.when(i + SA < nj)
                    def _():
                        gdesc(j0 + i + SA, s).start()

            return carry

        n_rounds = (per_w + 1 + SA - 1) // SA
        lax.fori_loop(0, n_rounds, body, 0)
        for s in range(SA):
            @pl.when((nj >= SA) | (s < nj))
            def _():
                wdesc(0, s).wait()

        if tail:
            @pl.when(wid == NW - 1)
            def _():
                pltpu.sync_copy(t5_hbm.at[:, pl.ds(n_full * 128, tail)], tbuf)

                @plsc.parallel_loop(0, tail, step=1, unroll=4)
                def _(l):
                    lfull = lane * 0 + l
                    row = l // 2
                    cb = (l % 2) * 64
                    for k in range(D // 16):
                        vals = plsc.load_gather(tbuf, [dvs[k], lfull])
                        bbuf[0, row, pl.ds(cb + k * 16, 16)] = vals

                pltpu.sync_copy(
                    bbuf.at[0, pl.ds(0, tail // 2)],
                    s2_hbm.at[pl.ds(n_full * 64, tail // 2)],
                )

    return pack


def _build_emb(H, B, V, D, NC, NS):
    NW = NC * NS
    BLK = B // NW
    CHUNK = BLK
    SLOTS = 4
    n_rounds = H // SLOTS
    assert H % SLOTS == 0 and BLK == 128

    mesh = plsc.VectorSubcoreMesh(core_axis_name="c", subcore_axis_name="s")

    @functools.partial(
        pl.kernel,
        mesh=mesh,
        out_type=jax.ShapeDtypeStruct((H, D // 8, NW, 8, BLK), jnp.float32),
        scratch_types=[
            pltpu.VMEM((H, BLK), jnp.int32),
            pltpu.VMEM((SLOTS, 2, BLK), jnp.int32),
            pltpu.VMEM((SLOTS, CHUNK, D // 2), jnp.float32),
            pltpu.VMEM((SLOTS, CHUNK, D // 2), jnp.float32),
            pltpu.VMEM((SLOTS, D // 8, 1, 8, BLK + 5), jnp.float32),
        ]
        + [pltpu.SemaphoreType.DMA] * (2 * SLOTS),
        compiler_params=pltpu.CompilerParams(
            use_tc_tiling_on_sc=False, needs_layout_passes=False
        ),
    )
    def emb(xt_hbm, s2q_hbm, out_hbm, idx_v, idq_v, rows_e, rows_o, tbuf_v, *sems):
        gsem = sems[:SLOTS]
        wsem = sems[SLOTS:]
        wid = lax.axis_index("s") * NC + lax.axis_index("c")
        base = wid * BLK
        pltpu.sync_copy(xt_hbm.at[:, pl.ds(base, BLK)], idx_v)

        lane = jax.lax.broadcasted_iota(jnp.int32, (16,), 0)
        lane_hi = lane // 8
        lane_lo = lane % 8
        zeros16 = lane * 0
        dhis = [lane_hi + 2 * k for k in range(D // 16)]

        def gdescs(h, j):
            return (
                pltpu.make_async_copy(
                    s2q_hbm.at[idq_v.at[j, 0]], rows_e.at[j], gsem[j]
                ),
                pltpu.make_async_copy(
                    s2q_hbm.at[idq_v.at[j, 1]], rows_o.at[j], gsem[j]
                ),
            )

        def wdesc(h, j):
            return pltpu.make_async_copy(
                tbuf_v.at[j, :, :, :, pl.ds(0, BLK)],
                out_hbm.at[h, pl.ds(0, D // 8), pl.ds(wid, 1)],
                wsem[j],
            )

        def stage_idx(h, j):
            @plsc.parallel_loop(0, BLK // 16, step=1, unroll=4)
            def _(g):
                v = idx_v[h, pl.ds(g * 16, 16)]
                v2 = v + v
                idq_v[j, 0, pl.ds(g * 16, 16)] = v2
                idq_v[j, 1, pl.ds(g * 16, 16)] = v2 + 1

        def fire(h, j):
            stage_idx(h, j)
            de, do = gdescs(h, j)
            de.start()
            do.start()

        for j in range(SLOTS):
            fire(j, j)

        def body(t, carry):
            for j in range(SLOTS):
                h = t * SLOTS + j
                de, do = gdescs(h, j)
                de.wait()
                do.wait()

                @pl.when(t > 0)
                def _():
                    wdesc(h - SLOTS, j).wait()

                @plsc.parallel_loop(0, CHUNK, step=1, unroll=8)
                def trans(b):
                    bfull = zeros16 + b
                    for k in range(D // 16):
                        src = rows_e if k < (D // 32) else rows_o
                        vals = src[j, b, pl.ds((k % (D // 32)) * 16, 16)]
                        plsc.store_scatter(
                            tbuf_v.at[j],
                            [dhis[k], zeros16, lane_lo, bfull],
                            vals,
                        )

                wdesc(h, j).start()

                @pl.when(h + SLOTS < H)
                def _():
                    fire(h + SLOTS, j)

            return carry

        lax.fori_loop(0, n_rounds, body, 0)
        for j in range(SLOTS):
            wdesc(H - SLOTS + j, j).wait()

    return emb


def kernel(x, table):
    Bt, H = x.shape
    V, D = table.shape
    info = plsc.get_sparse_core_info()
    pack = _build_pack(V, D, info.num_cores, info.num_subcores)
    emb = _build_emb(H, Bt, V, D, info.num_cores, info.num_subcores)
    s2 = pack(table.T)
    s2q = s2.reshape(s2.shape[0] * 4, D // 2)
    out5 = emb(x.T, s2q)
    return out5.transpose(2, 4, 0, 1, 3).reshape(Bt, H, D)


# speedup vs baseline: 7.1143x; 1.2661x over previous
import functools

import jax
import jax.numpy as jnp
from jax import lax
from jax.experimental import pallas as pl
from jax.experimental.pallas import tpu as pltpu
from jax.experimental.pallas import tpu_sc as plsc


def _build_emb(H, B, V, D, NC, NS):
    NW = NC * NS
    BLK = B // NW
    CHUNK = BLK
    SLOTS = 4
    n_rounds = H // SLOTS
    assert H % SLOTS == 0 and BLK == 128

    mesh = plsc.VectorSubcoreMesh(core_axis_name="c", subcore_axis_name="s")

    @functools.partial(
        pl.kernel,
        mesh=mesh,
        out_type=jax.ShapeDtypeStruct((H, D // 8, NW, 8, BLK), jnp.float32),
        scratch_types=[
            pltpu.VMEM((H, BLK), jnp.int32),
            pltpu.VMEM((SLOTS, CHUNK, D), jnp.float32),
            pltpu.VMEM((SLOTS, D // 8, 1, 8, BLK + 5), jnp.float32),
        ]
        + [pltpu.SemaphoreType.DMA] * (2 * SLOTS),
        compiler_params=pltpu.CompilerParams(
            use_tc_tiling_on_sc=False, needs_layout_passes=False
        ),
    )
    def emb(xt_hbm, table_hbm, out_hbm, idx_v, rows_v, tbuf_v, *sems):
        gsem = sems[:SLOTS]
        wsem = sems[SLOTS:]
        wid = lax.axis_index("s") * NC + lax.axis_index("c")
        base = wid * BLK
        pltpu.sync_copy(xt_hbm.at[:, pl.ds(base, BLK)], idx_v)

        lane = jax.lax.broadcasted_iota(jnp.int32, (16,), 0)
        lane_hi = lane // 8
        lane_lo = lane % 8
        zeros16 = lane * 0
        dhis = [lane_hi + 2 * k for k in range(D // 16)]

        def gdesc(h, j):
            return pltpu.make_async_copy(
                table_hbm.at[idx_v.at[h]], rows_v.at[j], gsem[j]
            )

        def wdesc(h, j):
            return pltpu.make_async_copy(
                tbuf_v.at[j, :, :, :, pl.ds(0, BLK)],
                out_hbm.at[h, pl.ds(0, D // 8), pl.ds(wid, 1)],
                wsem[j],
            )

        for j in range(SLOTS):
            gdesc(j, j).start()

        def body(t, carry):
            for j in range(SLOTS):
                h = t * SLOTS + j
                gdesc(h, j).wait()

                @pl.when(t > 0)
                def _():
                    wdesc(h - SLOTS, j).wait()

                @plsc.parallel_loop(0, CHUNK, step=1, unroll=8)
                def trans(b):
                    bfull = zeros16 + b
                    for k in range(D // 16):
                        vals = rows_v[j, b, pl.ds(k * 16, 16)]
                        plsc.store_scatter(
                            tbuf_v.at[j],
                            [dhis[k], zeros16, lane_lo, bfull],
                            vals,
                        )

                wdesc(h, j).start()

                @pl.when(h + SLOTS < H)
                def _():
                    gdesc(h + SLOTS, j).start()

            return carry

        lax.fori_loop(0, n_rounds, body, 0)
        for j in range(SLOTS):
            wdesc(H - SLOTS + j, j).wait()

    return emb


def kernel(x, table):
    Bt, H = x.shape
    V, D = table.shape
    info = plsc.get_sparse_core_info()
    emb = _build_emb(H, Bt, V, D, info.num_cores, info.num_subcores)
    out5 = emb(x.T, table)
    return out5.transpose(2, 4, 0, 1, 3).reshape(Bt, H, D)


# speedup vs baseline: 7.1225x; 1.0012x over previous
import functools

import jax
import jax.numpy as jnp
from jax import lax
from jax.experimental import pallas as pl
from jax.experimental.pallas import tpu as pltpu
from jax.experimental.pallas import tpu_sc as plsc


def _build_emb(H, B, V, D, NC, NS):
    NW = NC * NS
    BLK = B // NW
    CHUNK = BLK
    SLOTS = 5
    n_rounds = H // SLOTS
    assert H % SLOTS == 0 and BLK == 128

    mesh = plsc.VectorSubcoreMesh(core_axis_name="c", subcore_axis_name="s")

    @functools.partial(
        pl.kernel,
        mesh=mesh,
        out_type=jax.ShapeDtypeStruct((H, D // 8, NW, 8, BLK), jnp.float32),
        scratch_types=[
            pltpu.VMEM((H, BLK), jnp.int32),
            pltpu.VMEM((SLOTS, CHUNK, D), jnp.float32),
            pltpu.VMEM((SLOTS, D // 8, 1, 8, BLK + 5), jnp.float32),
        ]
        + [pltpu.SemaphoreType.DMA] * (2 * SLOTS),
        compiler_params=pltpu.CompilerParams(
            use_tc_tiling_on_sc=False, needs_layout_passes=False
        ),
    )
    def emb(xt_hbm, table_hbm, out_hbm, idx_v, rows_v, tbuf_v, *sems):
        gsem = sems[:SLOTS]
        wsem = sems[SLOTS:]
        wid = lax.axis_index("s") * NC + lax.axis_index("c")
        base = wid * BLK
        pltpu.sync_copy(xt_hbm.at[:, pl.ds(base, BLK)], idx_v)

        lane = jax.lax.broadcasted_iota(jnp.int32, (16,), 0)
        lane_hi = lane // 8
        lane_lo = lane % 8
        zeros16 = lane * 0
        dhis = [lane_hi + 2 * k for k in range(D // 16)]

        def gdesc(h, j):
            return pltpu.make_async_copy(
                table_hbm.at[idx_v.at[h]], rows_v.at[j], gsem[j]
            )

        def wdesc(h, j):
            return pltpu.make_async_copy(
                tbuf_v.at[j, :, :, :, pl.ds(0, BLK)],
                out_hbm.at[h, pl.ds(0, D // 8), pl.ds(wid, 1)],
                wsem[j],
            )

        for j in range(SLOTS):
            gdesc(j, j).start()

        def body(t, carry):
            for j in range(SLOTS):
                h = t * SLOTS + j
                gdesc(h, j).wait()

                @pl.when(t > 0)
                def _():
                    wdesc(h - SLOTS, j).wait()

                @plsc.parallel_loop(0, CHUNK, step=1, unroll=8)
                def trans(b):
                    bfull = zeros16 + b
                    for k in range(D // 16):
                        vals = rows_v[j, b, pl.ds(k * 16, 16)]
                        plsc.store_scatter(
                            tbuf_v.at[j],
                            [dhis[k], zeros16, lane_lo, bfull],
                            vals,
                        )

                wdesc(h, j).start()

                @pl.when(h + SLOTS < H)
                def _():
                    gdesc(h + SLOTS, j).start()

            return carry

        lax.fori_loop(0, n_rounds, body, 0)
        for j in range(SLOTS):
            wdesc(H - SLOTS + j, j).wait()

    return emb


def kernel(x, table):
    Bt, H = x.shape
    V, D = table.shape
    info = plsc.get_sparse_core_info()
    emb = _build_emb(H, Bt, V, D, info.num_cores, info.num_subcores)
    out5 = emb(x.T, table)
    return out5.transpose(2, 4, 0, 1, 3).reshape(Bt, H, D)
